# baseline (speedup 1.0000x reference)
import jax
import jax.numpy as jnp
from jax.experimental import pallas as pl
from jax.experimental.pallas import tpu as pltpu

_T, _N, _F, _TILE, _A, _K, _U = 16, 8, 6, 300, 21, 20, 100
_OUT = _TILE - _K + 1
_TNF = _T * _N * _F
_KA = _K * _A
_C = 512
_G = 20
_NG = _TILE // _G
_GU = _G * _U


def _r_kernel(p_ref, q_ref, r_ref):
    p = p_ref[...]
    m = jnp.max(p, axis=1, keepdims=True)
    lse = jnp.log(jnp.sum(jnp.exp(p - m), axis=1, keepdims=True)) + m
    logq = jnp.log(q_ref[...].reshape(_A))
    r_ref[...] = p - lse - logq[None, :, None]


def _conv_kernel(x_ref, wl_ref, wh_ref, o_ref):
    g = pl.program_id(0)
    off0 = pl.multiple_of(g * _C, _C)
    off1 = pl.multiple_of((g + 1) * _C, _C)
    x0 = x_ref[:, pl.ds(off0, _C)]
    x1 = x_ref[:, pl.ds(off1, _C)]
    acc = jnp.dot(x0, wl_ref[...], preferred_element_type=jnp.float32)
    acc += jnp.dot(x1, wh_ref[...], preferred_element_type=jnp.float32)
    o_ref[0] = acc


def _band_weights(r2):
    wl_cols, wh_cols = [], []
    for p in range(_G):
        s = _A * p
        wl_cols.append(jnp.pad(r2[:_KA - s], ((s, _C - _KA), (0, 0))))
        wh_cols.append(jnp.pad(r2[_KA - s:], ((0, _C - s), (0, 0))))
    wl = jnp.concatenate(wl_cols, axis=1)
    wh = jnp.concatenate(wh_cols, axis=1)
    return wl.astype(jnp.bfloat16), wh.astype(jnp.bfloat16)


def kernel(X, P_logit, Q):
    r3 = pl.pallas_call(
        _r_kernel,
        out_shape=jax.ShapeDtypeStruct((_K, _A, _U), jnp.float32),
    )(P_logit, Q.reshape(1, _A))
    wl, wh = _band_weights(r3.reshape(_KA, _U))

    x3 = X.reshape(_TNF, _NG, _KA)
    xp = jnp.pad(x3, ((0, 0), (0, 1), (0, _C - _KA)))
    xp = xp.reshape(_TNF, (_NG + 1) * _C).astype(jnp.bfloat16)

    z15 = pl.pallas_call(
        _conv_kernel,
        grid=(_NG,),
        in_specs=[
            pl.BlockSpec((_TNF, (_NG + 1) * _C), lambda g: (0, 0)),
            pl.BlockSpec((_C, _GU), lambda g: (0, 0)),
            pl.BlockSpec((_C, _GU), lambda g: (0, 0)),
        ],
        out_specs=pl.BlockSpec((1, _TNF, _GU), lambda g: (g, 0, 0)),
        out_shape=jax.ShapeDtypeStruct((_NG, _TNF, _GU), jnp.float32),
    )(xp, wl, wh)

    z = z15.transpose(1, 0, 2).reshape(_TNF, _TILE, _U)[:, :_OUT, :]
    return z.reshape(_T, _N, _F, _OUT, _U)


# speedup vs baseline: 7.5514x; 7.5514x over previous
import jax
import jax.numpy as jnp
from jax.experimental import pallas as pl

_T, _N, _F, _TILE, _A, _K, _U = 16, 8, 6, 300, 21, 20, 100
_OUT = _TILE - _K + 1
_TNF = _T * _N * _F
_KA = _K * _A
_C = 512
_G = 20
_NG = _TILE // _G
_UP = 128
_GU = _G * _UP
_BP = 128
_BM = 128


def _w_kernel(p_ref, q_ref, wl_ref, wh_ref):
    p = p_ref[...]
    m = jnp.max(p, axis=1, keepdims=True)
    lse = jnp.log(jnp.sum(jnp.exp(p - m), axis=1, keepdims=True)) + m
    logq = jnp.log(q_ref[...].reshape(_A))
    r2 = (p - lse - logq[None, :, None]).reshape(_KA, _U)
    wl_cols, wh_cols = [], []
    for i in range(_G):
        s = _A * i
        wl_cols.append(jnp.pad(r2[:_KA - s], ((s, _C - _KA), (0, _UP - _U))))
        if s == 0:
            wh_cols.append(jnp.zeros((_C, _UP), r2.dtype))
        else:
            wh_cols.append(jnp.pad(r2[_KA - s:], ((0, _C - s), (0, _UP - _U))))
    wl_ref[...] = jnp.concatenate(wl_cols, axis=1).astype(jnp.bfloat16)
    wh_ref[...] = jnp.concatenate(wh_cols, axis=1).astype(jnp.bfloat16)


def _prep_kernel(x_ref, o_ref):
    o_ref[...] = jnp.zeros_like(o_ref)
    x = x_ref[...].astype(jnp.bfloat16)
    for t in range(_TILE // 8):
        piece = x[:, 8 * t:8 * t + 8, :].reshape(_BP, 8 * _A)
        lo = 8 * _A * t
        hi = lo + 8 * _A
        g = lo // _KA
        split = min(hi, (g + 1) * _KA)
        o_ref[:, _C * g + lo - _KA * g:_C * g + split - _KA * g] = (
            piece[:, :split - lo])
        if split < hi:
            o_ref[:, _C * (g + 1):_C * (g + 1) + hi - split] = (
                piece[:, split - lo:])
    t8 = 8 * (_TILE // 8)
    tail = x[:, t8:, :].reshape(_BP, (_TILE - t8) * _A)
    lo = _A * t8
    o_ref[:, _C * (_NG - 1) + lo - _KA * (_NG - 1):
          _C * (_NG - 1) + lo - _KA * (_NG - 1) + (_TILE - t8) * _A] = tail


def _conv_kernel(x_ref, wl_ref, wh_ref, o_ref):
    wl = wl_ref[...]
    wh = wh_ref[...]
    for g in range(_NG):
        x0 = x_ref[:, _C * g:_C * (g + 1)]
        x1 = x_ref[:, _C * (g + 1):_C * (g + 2)]
        y = jnp.dot(x0, wl, preferred_element_type=jnp.float32)
        y += jnp.dot(x1, wh, preferred_element_type=jnp.float32)
        npos = _G if g < _NG - 1 else _OUT - _G * (_NG - 1)
        for i in range(npos):
            o_ref[:, _G * g + i, :] = y[:, _UP * i:_UP * i + _U]


def kernel(X, P_logit, Q):
    wl, wh = pl.pallas_call(
        _w_kernel,
        out_shape=(
            jax.ShapeDtypeStruct((_C, _GU), jnp.bfloat16),
            jax.ShapeDtypeStruct((_C, _GU), jnp.bfloat16),
        ),
    )(P_logit, Q.reshape(1, _A))

    x3 = X.reshape(_TNF, _TILE, _A)
    xp = pl.pallas_call(
        _prep_kernel,
        grid=(_TNF // _BP,),
        in_specs=[pl.BlockSpec((_BP, _TILE, _A), lambda r: (r, 0, 0))],
        out_specs=pl.BlockSpec((_BP, (_NG + 1) * _C), lambda r: (r, 0)),
        out_shape=jax.ShapeDtypeStruct((_TNF, (_NG + 1) * _C), jnp.bfloat16),
    )(x3)

    z = pl.pallas_call(
        _conv_kernel,
        grid=(_TNF // _BM,),
        in_specs=[
            pl.BlockSpec((_BM, (_NG + 1) * _C), lambda r: (r, 0)),
            pl.BlockSpec((_C, _GU), lambda r: (0, 0)),
            pl.BlockSpec((_C, _GU), lambda r: (0, 0)),
        ],
        out_specs=pl.BlockSpec((_BM, _OUT, _U), lambda r: (r, 0, 0)),
        out_shape=jax.ShapeDtypeStruct((_TNF, _OUT, _U), jnp.float32),
    )(xp, wl, wh)
    return z.reshape(_T, _N, _F, _OUT, _U)


# speedup vs baseline: 7.9784x; 1.0565x over previous
import jax
import jax.numpy as jnp
from jax.experimental import pallas as pl

_T, _N, _F, _TILE, _A, _K, _U = 16, 8, 6, 300, 21, 20, 100
_OUT = _TILE - _K + 1
_TNF = _T * _N * _F
_KA = _K * _A
_C = 512
_G = 20
_NG = _TILE // _G
_UP = 128
_GU = _G * _UP
_BT = 2
_BR = _BT * _N * _F


def _w_kernel(p_ref, q_ref, wl_ref, wh_ref):
    p = p_ref[...]
    m = jnp.max(p, axis=1, keepdims=True)
    lse = jnp.log(jnp.sum(jnp.exp(p - m), axis=1, keepdims=True)) + m
    logq = jnp.log(q_ref[...].reshape(_A))
    r2 = (p - lse - logq[None, :, None]).reshape(_KA, _U)
    wl_cols, wh_cols = [], []
    for i in range(_G):
        s = _A * i
        wl_cols.append(jnp.pad(r2[:_KA - s], ((s, _C - _KA), (0, _UP - _U))))
        if s == 0:
            wh_cols.append(jnp.zeros((_C, _UP), r2.dtype))
        else:
            wh_cols.append(jnp.pad(r2[_KA - s:], ((0, _C - s), (0, _UP - _U))))
    wl_ref[...] = jnp.concatenate(wl_cols, axis=1).astype(jnp.bfloat16)
    wh_ref[...] = jnp.concatenate(wh_cols, axis=1).astype(jnp.bfloat16)


def _prep_kernel(x_ref, o_ref):
    o_ref[...] = jnp.zeros_like(o_ref)
    x5 = x_ref[...].astype(jnp.bfloat16)
    x = x5.reshape(_BR, _TILE, _A)
    for t in range(_TILE // 8 + 1):
        q0 = 8 * t
        q1 = min(q0 + 8, _TILE)
        if q0 >= q1:
            break
        w = (q1 - q0) * _A
        piece = x[:, q0:q1, :].reshape(_BR, w)
        lo = _A * q0
        hi = lo + w
        g = lo // _KA
        split = min(hi, (g + 1) * _KA)
        o_ref[:, _C * g + lo - _KA * g:_C * g + split - _KA * g] = (
            piece[:, :split - lo])
        if split < hi:
            o_ref[:, _C * (g + 1):_C * (g + 1) + hi - split] = (
                piece[:, split - lo:])


def _conv_kernel(x_ref, wl_ref, wh_ref, o_ref):
    wl = wl_ref[...]
    wh = wh_ref[...]
    for g in range(_NG):
        x0 = x_ref[:, _C * g:_C * (g + 1)]
        x1 = x_ref[:, _C * (g + 1):_C * (g + 2)]
        y = jnp.dot(x0, wl, preferred_element_type=jnp.float32)
        y += jnp.dot(x1, wh, preferred_element_type=jnp.float32)
        npos = _G if g < _NG - 1 else _OUT - _G * (_NG - 1)
        for i in range(npos):
            piece = y[:, _UP * i:_UP * i + _U]
            o_ref[:, :, :, _G * g + i, :] = piece.reshape(_BT, _N, _F, _U)


def kernel(X, P_logit, Q):
    wl, wh = pl.pallas_call(
        _w_kernel,
        out_shape=(
            jax.ShapeDtypeStruct((_C, _GU), jnp.bfloat16),
            jax.ShapeDtypeStruct((_C, _GU), jnp.bfloat16),
        ),
    )(P_logit, Q.reshape(1, _A))

    xp = pl.pallas_call(
        _prep_kernel,
        grid=(_T // _BT,),
        in_specs=[pl.BlockSpec((_BT, _N, _F, _TILE, _A),
                               lambda r: (r, 0, 0, 0, 0))],
        out_specs=pl.BlockSpec((_BR, (_NG + 1) * _C), lambda r: (r, 0)),
        out_shape=jax.ShapeDtypeStruct((_TNF, (_NG + 1) * _C), jnp.bfloat16),
    )(X)

    z = pl.pallas_call(
        _conv_kernel,
        grid=(_T // _BT,),
        in_specs=[
            pl.BlockSpec((_BR, (_NG + 1) * _C), lambda r: (r, 0)),
            pl.BlockSpec((_C, _GU), lambda r: (0, 0)),
            pl.BlockSpec((_C, _GU), lambda r: (0, 0)),
        ],
        out_specs=pl.BlockSpec((_BT, _N, _F, _OUT, _U),
                               lambda r: (r, 0, 0, 0, 0)),
        out_shape=jax.ShapeDtypeStruct((_T, _N, _F, _OUT, _U), jnp.float32),
    )(xp, wl, wh)
    return z
